# Initial kernel scaffold; baseline (speedup 1.0000x reference)
import jax
import jax.numpy as jnp
from jax.experimental import pallas as pl


def kernel(x, batch_labels, batch_embed):
    raise NotImplementedError("write your pallas kernel here")


# speedup vs baseline: 1.1605x; 1.1605x over previous
import functools

import jax
import jax.numpy as jnp
from jax import lax
from jax.experimental import pallas as pl
from jax.experimental.pallas import tpu as pltpu
from jax.experimental.pallas import tpu_sc as plsc

EMBED_DIM = 64
NUM_BATCHES = 1000
B = 16384

NC = 2
NS = 16
NW = NC * NS
B_PER_W = B // NW
N_SUB = 4
SUB = B_PER_W // N_SUB


def _sc_body(x_hbm, idx_hbm, table_hbm, out_hbm, idx_v, rows_v, x_v, sem):
    wid = lax.axis_index("s") * NC + lax.axis_index("c")
    base = wid * B_PER_W

    for j in range(N_SUB):
        pltpu.sync_copy(idx_hbm.at[pl.ds(base + j * SUB, SUB)], idx_v.at[j])

    gathers = [
        pltpu.async_copy(table_hbm.at[idx_v.at[j]], rows_v.at[j], sem)
        for j in range(N_SUB)
    ]
    pltpu.sync_copy(x_hbm.at[pl.ds(base, B_PER_W)], x_v)
    for g in gathers:
        g.wait()

    def sub_row(r, _):
        for c in range(EMBED_DIM // 16):
            sl = pl.ds(c * 16, 16)
            x_v[r, sl] = x_v[r, sl] - rows_v[r // SUB, r % SUB, sl]
        return 0

    lax.fori_loop(0, B_PER_W, sub_row, 0)

    pltpu.sync_copy(x_v, out_hbm.at[pl.ds(base, B_PER_W)])


@jax.jit
def _batch_correct(x, batch_labels, batch_embed):
    mesh = plsc.VectorSubcoreMesh(core_axis_name="c", subcore_axis_name="s")
    return pl.kernel(
        _sc_body,
        out_type=jax.ShapeDtypeStruct((B, EMBED_DIM), jnp.float32),
        mesh=mesh,
        scratch_types=[
            pltpu.VMEM((N_SUB, SUB), jnp.int32),
            pltpu.VMEM((N_SUB, SUB, EMBED_DIM), jnp.float32),
            pltpu.VMEM((B_PER_W, EMBED_DIM), jnp.float32),
            pltpu.SemaphoreType.DMA,
        ],
        compiler_params=pltpu.CompilerParams(use_tc_tiling_on_sc=False),
    )(x, batch_labels, batch_embed)


def kernel(x, batch_labels, batch_embed):
    return _batch_correct(x, batch_labels.astype(jnp.int32), batch_embed)


# speedup vs baseline: 1.2141x; 1.0462x over previous
import jax
import jax.numpy as jnp
from jax import lax
from jax.experimental import pallas as pl
from jax.experimental.pallas import tpu as pltpu
from jax.experimental.pallas import tpu_sc as plsc

EMBED_DIM = 64
NUM_BATCHES = 1000
B = 16384

NC = 2
NS = 16
NW = NC * NS
B_PER_W = B // NW
N_SUB = 4
SUB = B_PER_W // N_SUB


def _sc_body(x_hbm, idx_hbm, table_hbm, out_hbm,
             idx_v, rows_v, x_v, x_sem, g_sems, o_sem):
    wid = lax.axis_index("s") * NC + lax.axis_index("c")
    base = wid * B_PER_W

    x_copy = pltpu.async_copy(x_hbm.at[pl.ds(base, B_PER_W)], x_v, x_sem)
    pltpu.sync_copy(idx_hbm.at[wid], idx_v)
    gathers = [
        pltpu.async_copy(table_hbm.at[idx_v.at[j]], rows_v.at[j], g_sems.at[j])
        for j in range(N_SUB)
    ]
    x_copy.wait()

    stores = []
    for j in range(N_SUB):
        gathers[j].wait()

        def sub_row(r, _):
            for c in range(EMBED_DIM // 16):
                sl = pl.ds(c * 16, 16)
                x_v[j * SUB + r, sl] = x_v[j * SUB + r, sl] - rows_v[j, r, sl]
            return 0

        lax.fori_loop(0, SUB, sub_row, 0)
        stores.append(pltpu.async_copy(
            x_v.at[pl.ds(j * SUB, SUB)],
            out_hbm.at[pl.ds(base + j * SUB, SUB)], o_sem))
    for s in stores:
        s.wait()


@jax.jit
def _batch_correct(x, batch_labels, batch_embed):
    mesh = plsc.VectorSubcoreMesh(core_axis_name="c", subcore_axis_name="s")
    idx3 = batch_labels.reshape(NW, N_SUB, SUB)
    return pl.kernel(
        _sc_body,
        out_type=jax.ShapeDtypeStruct((B, EMBED_DIM), jnp.float32),
        mesh=mesh,
        scratch_types=[
            pltpu.VMEM((N_SUB, SUB), jnp.int32),
            pltpu.VMEM((N_SUB, SUB, EMBED_DIM), jnp.float32),
            pltpu.VMEM((B_PER_W, EMBED_DIM), jnp.float32),
            pltpu.SemaphoreType.DMA,
            pltpu.SemaphoreType.DMA((N_SUB,)),
            pltpu.SemaphoreType.DMA,
        ],
        compiler_params=pltpu.CompilerParams(
            use_tc_tiling_on_sc=False,
            disable_bounds_checks=True,
            disable_semaphore_checks=True,
            skip_device_barrier=True,
        ),
    )(x, idx3, batch_embed)


def kernel(x, batch_labels, batch_embed):
    return _batch_correct(x, batch_labels.astype(jnp.int32), batch_embed)
